# Initial kernel scaffold; baseline (speedup 1.0000x reference)
import functools

import jax
import jax.numpy as jnp
from jax import lax
from jax.experimental import pallas as pl
from jax.experimental.pallas import tpu as pltpu
from jax.experimental.pallas import tpu_sc as plsc

BATCH = 16384
HIST = 50
D = 32
B = BATCH * HIST
NC = 2
NS = 16
NW = NC * NS
ROWS_PER_W = B // NW
IPR = 128
G = 10
CHUNK = G * IPR
NCHUNK = ROWS_PER_W // CHUNK
IDX_ROWS_PER_W = ROWS_PER_W // IPR


def _embed_call(idx2d, table):
    mesh = plsc.VectorSubcoreMesh(core_axis_name="c", subcore_axis_name="s")

    @functools.partial(
        pl.kernel,
        mesh=mesh,
        out_type=jax.ShapeDtypeStruct((B, D), jnp.float32),
        scratch_types=[
            pltpu.VMEM((IDX_ROWS_PER_W, IPR), jnp.int32),
            pltpu.VMEM((2, CHUNK, D), jnp.float32),
            pltpu.SemaphoreType.DMA,
            pltpu.SemaphoreType.DMA,
            pltpu.SemaphoreType.DMA,
            pltpu.SemaphoreType.DMA,
        ],
    )
    def k(idx_hbm, table_hbm, out_hbm, idx_v, rows_v, g0, g1, s0, s1):
        wid = lax.axis_index("s") * NC + lax.axis_index("c")
        idx_base = wid * IDX_ROWS_PER_W
        out_base = wid * ROWS_PER_W
        pltpu.sync_copy(idx_hbm.at[pl.ds(idx_base, IDX_ROWS_PER_W)], idx_v)

        gsems = (g0, g1)
        ssems = (s0, s1)

        def issue_gathers(c, b):
            for j in range(G):
                pltpu.async_copy(
                    table_hbm.at[idx_v.at[c * G + j]],
                    rows_v.at[b, pl.ds(j * IPR, IPR)],
                    gsems[b],
                )

        def drain_gathers(b):
            pltpu.make_async_copy(
                table_hbm.at[pl.ds(0, CHUNK)], rows_v.at[b], gsems[b]
            ).wait()

        def start_store(c, b):
            pltpu.async_copy(
                rows_v.at[b],
                out_hbm.at[pl.ds(out_base + c * CHUNK, CHUNK)],
                ssems[b],
            )

        def drain_store(b):
            pltpu.make_async_copy(
                rows_v.at[b], out_hbm.at[pl.ds(out_base, CHUNK)], ssems[b]
            ).wait()

        issue_gathers(0, 0)
        issue_gathers(1, 1)

        def step(s, carry):
            for b in range(2):
                c = s * 2 + b
                drain_gathers(b)
                start_store(c, b)

                @pl.when(s < NCHUNK // 2 - 1)
                def _():
                    drain_store(b)
                    issue_gathers(c + 2, b)

            return carry

        lax.fori_loop(0, NCHUNK // 2, step, 0)
        drain_store(0)
        drain_store(1)

    return k(idx2d, table)


def kernel(inputs, table):
    idx2d = inputs.reshape(B // IPR, IPR)
    out = _embed_call(idx2d, table)
    return out.reshape(BATCH, HIST, D)


# speedup vs baseline: 1.1128x; 1.1128x over previous
import functools

import jax
import jax.numpy as jnp
from jax import lax
from jax.experimental import pallas as pl
from jax.experimental.pallas import tpu as pltpu
from jax.experimental.pallas import tpu_sc as plsc

BATCH = 16384
HIST = 50
D = 32
B = BATCH * HIST
NC = 2
NS = 16
NW = NC * NS
ROWS_PER_W = B // NW
IPR = 128
G = 10
CHUNK = G * IPR
NCHUNK = ROWS_PER_W // CHUNK
IDX_ROWS_PER_W = ROWS_PER_W // IPR


def _embed_call(idx2d, table):
    mesh = plsc.VectorSubcoreMesh(core_axis_name="c", subcore_axis_name="s")

    @functools.partial(
        pl.kernel,
        mesh=mesh,
        out_type=jax.ShapeDtypeStruct((B, D), jnp.float32),
        compiler_params=pltpu.CompilerParams(use_tc_tiling_on_sc=False),
        scratch_types=[
            pltpu.VMEM((IDX_ROWS_PER_W, IPR), jnp.int32),
            pltpu.VMEM((2, CHUNK, D), jnp.float32),
            pltpu.SemaphoreType.DMA,
            pltpu.SemaphoreType.DMA,
            pltpu.SemaphoreType.DMA,
            pltpu.SemaphoreType.DMA,
        ],
    )
    def k(idx_hbm, table_hbm, out_hbm, idx_v, rows_v, g0, g1, s0, s1):
        wid = lax.axis_index("s") * NC + lax.axis_index("c")
        idx_base = wid * IDX_ROWS_PER_W
        out_base = wid * ROWS_PER_W
        pltpu.sync_copy(idx_hbm.at[pl.ds(idx_base, IDX_ROWS_PER_W)], idx_v)

        gsems = (g0, g1)
        ssems = (s0, s1)

        def issue_gathers(c, b):
            for j in range(G):
                pltpu.async_copy(
                    table_hbm.at[idx_v.at[c * G + j]],
                    rows_v.at[b, pl.ds(j * IPR, IPR)],
                    gsems[b],
                )

        def drain_gathers(b):
            pltpu.make_async_copy(
                table_hbm.at[pl.ds(0, CHUNK)], rows_v.at[b], gsems[b]
            ).wait()

        def start_store(c, b):
            pltpu.async_copy(
                rows_v.at[b],
                out_hbm.at[pl.ds(out_base + c * CHUNK, CHUNK)],
                ssems[b],
            )

        def drain_store(b):
            pltpu.make_async_copy(
                rows_v.at[b], out_hbm.at[pl.ds(out_base, CHUNK)], ssems[b]
            ).wait()

        issue_gathers(0, 0)
        issue_gathers(1, 1)

        def step(s, carry):
            for b in range(2):
                c = s * 2 + b
                drain_gathers(b)
                start_store(c, b)

                @pl.when(s < NCHUNK // 2 - 1)
                def _():
                    drain_store(b)
                    issue_gathers(c + 2, b)

            return carry

        lax.fori_loop(0, NCHUNK // 2, step, 0)
        drain_store(0)
        drain_store(1)

    return k(idx2d, table)


def kernel(inputs, table):
    idx2d = inputs.reshape(B // IPR, IPR)
    out = _embed_call(idx2d, table)
    return out.reshape(BATCH, HIST, D)


# speedup vs baseline: 1.6439x; 1.4773x over previous
import functools

import jax
import jax.numpy as jnp
from jax import lax
from jax.experimental import pallas as pl
from jax.experimental.pallas import tpu as pltpu
from jax.experimental.pallas import tpu_sc as plsc

BATCH = 16384
HIST = 50
D = 32
NC = 2
NS = 16
NW = NC * NS
IPR = 128
BT_PER_W = (BATCH // IPR) // NW
UNITS = HIST * BT_PER_W
DHI = D // 8


def _embed_call(idx_t, table):
    mesh = plsc.VectorSubcoreMesh(core_axis_name="c", subcore_axis_name="s")

    @functools.partial(
        pl.kernel,
        mesh=mesh,
        out_type=jax.ShapeDtypeStruct((HIST, DHI, BATCH // IPR, 8, IPR),
                                      jnp.float32),
        compiler_params=pltpu.CompilerParams(
            use_tc_tiling_on_sc=False, needs_layout_passes=False
        ),
        scratch_types=[
            pltpu.VMEM((HIST, BT_PER_W * IPR), jnp.int32),
            pltpu.VMEM((2, IPR, D), jnp.float32),
            pltpu.VMEM((2, D, IPR), jnp.float32),
            pltpu.SemaphoreType.DMA,
            pltpu.SemaphoreType.DMA,
            pltpu.SemaphoreType.DMA,
            pltpu.SemaphoreType.DMA,
        ],
    )
    def k(idx_hbm, table_hbm, out_hbm, idx_v, rows_v, tsp_v, g0, g1, s0, s1):
        wid = lax.axis_index("s") * NC + lax.axis_index("c")
        bt_base = wid * BT_PER_W
        pltpu.sync_copy(idx_hbm.at[:, pl.ds(bt_base * IPR, BT_PER_W * IPR)],
                        idx_v)

        gsems = (g0, g1)
        ssems = (s0, s1)
        lane = lax.iota(jnp.int32, 16)
        row_bases = [lane + 16 * kk for kk in range(8)]

        def issue_gather(u, b):
            h = u // BT_PER_W
            t = u % BT_PER_W
            pltpu.async_copy(
                table_hbm.at[idx_v.at[h, pl.ds(t * IPR, IPR)]],
                rows_v.at[b],
                gsems[b],
            )

        def drain_gather(b):
            pltpu.make_async_copy(
                table_hbm.at[pl.ds(0, IPR)], rows_v.at[b], gsems[b]
            ).wait()

        def transpose(b):
            def body(d, carry):
                col = jnp.full((16,), 0, jnp.int32) + d
                for kk in range(8):
                    v = plsc.load_gather(rows_v.at[b], [row_bases[kk], col])
                    tsp_v[b, d, pl.ds(16 * kk, 16)] = v
                return carry

            lax.fori_loop(0, D, body, 0)

        def issue_stores(u, b):
            h = u // BT_PER_W
            t = u % BT_PER_W
            for kk in range(DHI):
                pltpu.async_copy(
                    tsp_v.at[b, pl.ds(8 * kk, 8)],
                    out_hbm.at[h, kk, bt_base + t],
                    ssems[b],
                )

        def drain_stores(b):
            for kk in range(DHI):
                pltpu.make_async_copy(
                    tsp_v.at[b, pl.ds(8 * kk, 8)], out_hbm.at[0, kk, 0],
                    ssems[b],
                ).wait()

        issue_gather(0, 0)
        issue_gather(1, 1)

        def pair(s, carry):
            for b in range(2):
                u = s * 2 + b
                drain_gather(b)

                @pl.when(s >= 1)
                def _():
                    drain_stores(b)

                transpose(b)
                issue_stores(u, b)

                @pl.when(s < UNITS // 2 - 1)
                def _():
                    issue_gather(u + 2, b)

            return carry

        lax.fori_loop(0, UNITS // 2, pair, 0)
        drain_stores(0)
        drain_stores(1)

    return k(idx_t, table)


def kernel(inputs, table):
    idx_t = inputs.T
    o6 = _embed_call(idx_t, table)
    return o6.transpose(2, 4, 0, 1, 3).reshape(BATCH, HIST, D)
